# baseline (speedup 1.0000x reference)
import jax
import jax.numpy as jnp
from jax import lax
from jax.experimental import pallas as pl
from jax.experimental.pallas import tpu as pltpu
from jax.experimental.pallas import tpu_sc as plsc

NC = 2
NS = 16
LANE = 16
HALF = 16
CHUNK = 256
BLK = 2
EB = CHUNK * BLK
UB = 128
LAYERS = 3
DECAY = 0.8
SLOPE = 0.8


def _ceil_to(x, m):
    return ((x + m - 1) // m) * m


def _make_fused(n_s, e_s, n_e, e_e, n_k, e_k, n_ids):
    nr_s = _ceil_to(n_s + 1, NS * UB)
    nr_e = _ceil_to(n_e + 1, NS * UB)
    nr_k = _ceil_to(n_k + 1, NS * UB)
    stride = nr_s

    mesh = plsc.VectorSubcoreMesh(core_axis_name="c", subcore_axis_name="s",
                                  num_cores=NC, num_subcores=NS)
    out_type = [
        jax.ShapeDtypeStruct((NC * stride, HALF), jnp.float32),
        jax.ShapeDtypeStruct((NC * stride, HALF), jnp.float32),
        jax.ShapeDtypeStruct((NC * n_ids, HALF), jnp.float32),
        jax.ShapeDtypeStruct((NC * n_ids, HALF), jnp.float32),
    ]
    scratch = [
        pltpu.VMEM_SHARED((stride, HALF), jnp.float32),
        pltpu.VMEM((BLK, CHUNK), jnp.int32),
        pltpu.VMEM((BLK, CHUNK), jnp.int32),
        pltpu.VMEM((BLK, CHUNK), jnp.int32),
        pltpu.VMEM((BLK, CHUNK), jnp.int32),
        pltpu.VMEM((BLK, CHUNK, HALF), jnp.float32),
        pltpu.VMEM((BLK, CHUNK, HALF), jnp.float32),
        pltpu.VMEM((UB, HALF), jnp.float32),
        pltpu.VMEM((UB, HALF), jnp.float32),
        pltpu.VMEM((UB, HALF), jnp.float32),
        pltpu.VMEM((UB, HALF), jnp.float32),
        pltpu.VMEM((UB,), jnp.int32),
        pltpu.VMEM((4, LANE), jnp.float32),
    ] + [pltpu.SemaphoreType.DMA] * 10

    def body(t_s, col_s, row_s, id_s, t_e, col_e, row_e, id_e,
             t_k, col_k, row_k, scales,
             emb_o, sum_o, gath_s, gath_e,
             agg, colv0, colv1, rowv0, rowv1, rbuf0, rbuf1,
             aggv, ev, sv, zv, idxu, sclv,
             g0, g1, g2, g3, g4, g5, g6, g7, ssem0, ssem1):
        banks = ((colv0, rowv0, rbuf0, ssem0, (g0, g1)),
                 (colv1, rowv1, rbuf1, ssem1, (g2, g3)))
        c = lax.axis_index("c")
        s = lax.axis_index("s")

        pltpu.sync_copy(scales, sclv)

        def zf(r, carry):
            zv[r] = jnp.zeros((LANE,), jnp.float32)
            return carry
        lax.fori_loop(0, UB, zf, 0)

        rpt_all = stride // NS
        def zb(b, carry):
            pltpu.async_copy(zv, agg.at[pl.ds(s * rpt_all + b * UB, UB)], g4)
            return carry
        lax.fori_loop(0, rpt_all // UB, zb, 0)
        for _ in range(rpt_all // UB):
            pltpu.make_async_copy(zv, agg.at[pl.ds(0, UB)], g4).wait()
        plsc.subcore_barrier()

        def run_graph(gi, table, colr, rowr, idsr, gath_o, n_nodes, n_rnd,
                      e_pad):
            sc_vec = sclv[gi]
            nblk = e_pad // (NS * EB)
            rpt = n_rnd // NS
            ublk = rpt // UB

            for l in range(LAYERS):
                src = table if l == 0 else emb_o

                def rebase(cv):
                    if l == 0:
                        mul, off = 2, c
                    else:
                        mul, off = 1, c * stride
                    for j in range(BLK):
                        for i in range(CHUNK // LANE):
                            sl = pl.ds(i * LANE, LANE)
                            cv[j, sl] = cv[j, sl] * mul + off

                def sbody(i, carry):
                    for p, (cv, rv, rb_, ssem, gs) in enumerate(banks):
                        b = 2 * i + p
                        offc = (s * nblk + b) * BLK

                        @pl.when(i >= 1)
                        def _drain():
                            for j in range(BLK):
                                pltpu.make_async_copy(
                                    rb_.at[j], agg.at[rv.at[j]], ssem).wait()

                        pltpu.sync_copy(colr.at[pl.ds(offc, BLK)], cv)
                        pltpu.sync_copy(rowr.at[pl.ds(offc, BLK)], rv)
                        rebase(cv)
                        descs = [
                            pltpu.async_copy(src.at[cv.at[j]], rb_.at[j],
                                             gs[j])
                            for j in range(BLK)]
                        for j in range(BLK):
                            descs[j].wait()
                            pltpu.async_copy(rb_.at[j], agg.at[rv.at[j]],
                                             ssem, add=True)
                    return carry
                lax.fori_loop(0, nblk // 2, sbody, 0)
                for (cv, rv, rb_, ssem, gs) in banks:
                    for j in range(BLK):
                        pltpu.make_async_copy(rb_.at[j], agg.at[rv.at[j]],
                                              ssem).wait()
                plsc.subcore_barrier()

                def ubody(b, carry):
                    lo = s * rpt + b * UB
                    glo = c * stride + lo

                    @pl.when(b >= 1)
                    def _drain_stores():
                        pltpu.make_async_copy(
                            ev, emb_o.at[pl.ds(glo, UB)], g6).wait()
                        pltpu.make_async_copy(
                            sv, sum_o.at[pl.ds(glo, UB)], g7).wait()
                        pltpu.make_async_copy(
                            zv, agg.at[pl.ds(lo, UB)], g5).wait()

                    la = pltpu.async_copy(agg.at[pl.ds(lo, UB)], aggv, g0)
                    if l == 0:
                        base = 2 * lo + c
                        cap = 2 * n_nodes - 1
                        for i in range(UB // LANE):
                            sl = pl.ds(i * LANE, LANE)
                            idxu[sl] = jnp.minimum(
                                2 * lax.iota(jnp.int32, LANE)
                                + (2 * i * LANE + base), cap)
                        lb = pltpu.async_copy(src.at[idxu], ev, g1)
                    else:
                        lb = pltpu.async_copy(src.at[pl.ds(glo, UB)], ev, g1)
                    if l > 0:
                        lc = pltpu.async_copy(sum_o.at[pl.ds(glo, UB)], sv, g2)
                        lc.wait()
                    la.wait()
                    lb.wait()
                    pltpu.async_copy(zv, agg.at[pl.ds(lo, UB)], g5)

                    def rb(r, cc):
                        a = aggv[r]
                        e = ev[r]
                        nw = sc_vec * a + DECAY * e
                        ev[r] = nw
                        if l == 0:
                            sv[r] = e + nw
                        else:
                            sv[r] = sv[r] + nw
                        return cc
                    lax.fori_loop(0, UB, rb, 0)
                    pltpu.async_copy(ev, emb_o.at[pl.ds(glo, UB)], g6)
                    pltpu.async_copy(sv, sum_o.at[pl.ds(glo, UB)], g7)
                    return carry
                lax.fori_loop(0, ublk, ubody, 0)
                pltpu.make_async_copy(ev, emb_o.at[pl.ds(0, UB)], g6).wait()
                pltpu.make_async_copy(sv, sum_o.at[pl.ds(0, UB)], g7).wait()
                pltpu.make_async_copy(zv, agg.at[pl.ds(0, UB)], g5).wait()
                plsc.subcore_barrier()

            if idsr is not None:
                ipt = n_ids // NS // CHUNK
                for h in range(ipt // BLK):
                    cv, rv, rb_, ssem, gs = banks[h % 2]
                    pltpu.sync_copy(idsr.at[s * (ipt // BLK) + h], cv)
                    for j in range(BLK):
                        for i in range(CHUNK // LANE):
                            sl = pl.ds(i * LANE, LANE)
                            cv[j, sl] = cv[j, sl] + c * stride
                    descs = [
                        pltpu.async_copy(sum_o.at[cv.at[j]], rb_.at[j], gs[j])
                        for j in range(BLK)]
                    for j in range(BLK):
                        descs[j].wait()
                        pltpu.sync_copy(
                            rb_.at[j],
                            gath_o.at[pl.ds(
                                c * n_ids + (s * ipt + h * BLK + j) * CHUNK,
                                CHUNK)])
                plsc.subcore_barrier()

        run_graph(0, t_s, col_s, row_s, id_s, gath_s, n_s, nr_s, e_s)
        run_graph(1, t_e, col_e, row_e, id_e, gath_e, n_e, nr_e, e_e)
        run_graph(2, t_k, col_k, row_k, None, None, n_k, nr_k, e_k)

    return pl.kernel(
        body, out_type=out_type, scratch_types=scratch, mesh=mesh,
        compiler_params=pltpu.CompilerParams(use_tc_tiling_on_sc=False),
    ), stride, (nr_s, nr_e, nr_k)


def _prep_edges(row, col, n_nodes):
    e = row.shape[0]
    e_pad = _ceil_to(e, 2 * NS * EB)
    colr = jnp.pad(col, (0, e_pad - e)).reshape(e_pad // CHUNK, CHUNK)
    rowr = jnp.pad(row, (0, e_pad - e),
                   constant_values=n_nodes).reshape(e_pad // CHUNK, CHUNK)
    return colr, rowr, e_pad


def _conv_all(s_tab, s_row, s_col, s_val, sid,
              e_tab, e_row, e_col, e_val, eid,
              k_tab, k_row, k_col, k_val):
    n_s, n_e, n_k = s_tab.shape[0], e_tab.shape[0], k_tab.shape[0]
    n_ids = sid.shape[0]
    col_s, row_s, e_s = _prep_edges(s_row, s_col, n_s)
    col_e, row_e, e_e = _prep_edges(e_row, e_col, n_e)
    col_k, row_k, e_k = _prep_edges(k_row, k_col, n_k)
    fused, stride, _ = _make_fused(n_s, e_s, n_e, e_e, n_k, e_k, n_ids)

    t_s = s_tab.reshape(NC * n_s, HALF)
    t_e = e_tab.reshape(NC * n_e, HALF)
    t_k = k_tab.reshape(NC * n_k, HALF)
    id_s = sid.reshape(n_ids // (BLK * CHUNK), BLK, CHUNK)
    id_e = eid.reshape(n_ids // (BLK * CHUNK), BLK, CHUNK)
    scales = jnp.stack([
        jnp.broadcast_to(s_val[0], (LANE,)),
        jnp.broadcast_to(e_val[0], (LANE,)),
        jnp.broadcast_to(k_val[0], (LANE,)),
        jnp.zeros((LANE,), jnp.float32)])

    _, ssum, gs, ge = fused(t_s, col_s, row_s, id_s, t_e, col_e, row_e, id_e,
                            t_k, col_k, row_k, scales)
    ems = gs.reshape(NC, n_ids, HALF).transpose(1, 0, 2).reshape(n_ids, 2 * HALF)
    eme = ge.reshape(NC, n_ids, HALF).transpose(1, 0, 2).reshape(n_ids, 2 * HALF)
    ck = (ssum.reshape(NC, stride, HALF)[:, :n_k]
          .transpose(1, 0, 2).reshape(n_k, 2 * HALF))
    return ems, eme, ck


RB = 2048


def _head(ems, eme, kn, ck, Ws, bS, We_, bE, Wk, bK, Wd, bD,
          W1, b1, W2, b2, W3, b3, W4, b4):
    b = ems.shape[0]
    k_num = ck.shape[0]

    def body(ems_r, eme_r, kn_r, ck_r, ws_r, bs_r, we_r, be_r, wk_r, bk_r,
             wd_r, bd_r, w1_r, b1_r, w2_r, b2_r, w3_r, b3_r, w4_r, b4_r,
             out_r):
        def dot_t(x, w):
            return lax.dot_general(x, w, (((1,), (1,)), ((), ())),
                                   preferred_element_type=jnp.float32)

        def lrelu(x):
            return jnp.where(x > 0, x, SLOPE * x)

        es = ems_r[...] * 0.25
        ee = eme_r[...] * 0.25
        ckv = ck_r[...] * 0.25
        sf = lrelu(dot_t(es, ws_r[...]) + bs_r[...])
        ef = lrelu(dot_t(ee, we_r[...]) + be_r[...])
        kf = lrelu(dot_t(ckv, wk_r[...]) + bk_r[...])
        disc = jax.nn.sigmoid(
            jnp.sum(ee * wd_r[...], axis=1, keepdims=True) + bd_r[...])
        st = disc * dot_t(sf - ef, kf) * kn_r[...]
        h = jnp.tanh(dot_t(st, w1_r[...]) + b1_r[...])
        h = jnp.tanh(dot_t(h, w2_r[...]) + b2_r[...])
        h = jnp.tanh(dot_t(h, w3_r[...]) + b3_r[...])
        o = jnp.sum(h * w4_r[...], axis=1, keepdims=True) + b4_r[...]
        out_r[...] = jax.nn.sigmoid(o)

    full = lambda shp: pl.BlockSpec(shp, lambda i: (0, 0))
    grid = b // RB
    return pl.pallas_call(
        body,
        grid=(grid,),
        in_specs=[
            pl.BlockSpec((RB, 2 * HALF), lambda i: (i, 0)),
            pl.BlockSpec((RB, 2 * HALF), lambda i: (i, 0)),
            pl.BlockSpec((RB, k_num), lambda i: (i, 0)),
            full(ck.shape), full(Ws.shape), full((1, Ws.shape[0])),
            full(We_.shape), full((1, We_.shape[0])),
            full(Wk.shape), full((1, Wk.shape[0])),
            full(Wd.shape), full((1, 1)),
            full(W1.shape), full((1, W1.shape[0])),
            full(W2.shape), full((1, W2.shape[0])),
            full(W3.shape), full((1, W3.shape[0])),
            full(W4.shape), full((1, 1)),
        ],
        out_specs=pl.BlockSpec((RB, 1), lambda i: (i, 0)),
        out_shape=jax.ShapeDtypeStruct((b, 1), jnp.float32),
    )(ems, eme, kn, ck, Ws, bS.reshape(1, -1), We_, bE.reshape(1, -1),
      Wk, bK.reshape(1, -1), Wd, bD.reshape(1, 1),
      W1, b1.reshape(1, -1), W2, b2.reshape(1, -1),
      W3, b3.reshape(1, -1), W4, b4.reshape(1, 1))


def kernel(student_id, exercise_id, knowledge,
           s_row, s_col, s_val, e_row, e_col, e_val, k_row, k_col, k_val,
           student_table, exercise_table, knowledge_table,
           Ws, bS, We_, bE, Wk, bK, Wd, bD,
           W1, b1, W2, b2, W3, b3, W4, b4):
    ems, eme, ck = _conv_all(student_table, s_row, s_col, s_val, student_id,
                             exercise_table, e_row, e_col, e_val, exercise_id,
                             knowledge_table, k_row, k_col, k_val)
    out = _head(ems, eme, knowledge, ck, Ws, bS, We_, bE, Wk, bK, Wd, bD,
                W1, b1, W2, b2, W3, b3, W4, b4)
    return out.reshape(-1)


# speedup vs baseline: 1.1153x; 1.1153x over previous
import jax
import jax.numpy as jnp
from jax import lax
from jax.experimental import pallas as pl
from jax.experimental.pallas import tpu as pltpu
from jax.experimental.pallas import tpu_sc as plsc

NC = 2
NS = 16
LANE = 16
HALF = 16
CHUNK = 256
BLK = 2
EB = CHUNK * BLK
UB = 128
LAYERS = 3
DECAY = 0.8
SLOPE = 0.8


def _ceil_to(x, m):
    return ((x + m - 1) // m) * m


def _make_fused(n_s, e_s, n_e, e_e, n_k, e_k, n_ids):
    nr_s = _ceil_to(n_s + 1, NS * UB)
    nr_e = _ceil_to(n_e + 1, NS * UB)
    nr_k = _ceil_to(n_k + 1, NS * UB)
    stride = nr_s

    mesh = plsc.VectorSubcoreMesh(core_axis_name="c", subcore_axis_name="s",
                                  num_cores=NC, num_subcores=NS)
    out_type = [
        jax.ShapeDtypeStruct((NC * stride, HALF), jnp.float32),
        jax.ShapeDtypeStruct((NC * stride, HALF), jnp.float32),
        jax.ShapeDtypeStruct((NC * n_ids, HALF), jnp.float32),
        jax.ShapeDtypeStruct((NC * n_ids, HALF), jnp.float32),
    ]
    scratch = [
        pltpu.VMEM_SHARED((stride, HALF), jnp.float32),
        pltpu.VMEM((BLK, 2, CHUNK), jnp.int32),
        pltpu.VMEM((BLK, 2, CHUNK), jnp.int32),
        pltpu.VMEM((BLK, CHUNK, HALF), jnp.float32),
        pltpu.VMEM((BLK, CHUNK, HALF), jnp.float32),
        pltpu.VMEM((UB, HALF), jnp.float32),
        pltpu.VMEM((UB, HALF), jnp.float32),
        pltpu.VMEM((UB, HALF), jnp.float32),
        pltpu.VMEM((UB, HALF), jnp.float32),
        pltpu.VMEM((4, LANE), jnp.float32),
    ] + [pltpu.SemaphoreType.DMA] * 10

    def body(t_s, ci_s, id_s, t_e, ci_e, id_e, t_k, ci_k, scales,
             emb_o, sum_o, gath_s, gath_e,
             agg, civ0, civ1, rbuf0, rbuf1, aggv, ev, sv, zv, sclv,
             g0, g1, g2, g3, g4, g5, g6, g7, ssem0, ssem1):
        banks = ((civ0, rbuf0, ssem0, (g0, g1)),
                 (civ1, rbuf1, ssem1, (g2, g3)))
        c = lax.axis_index("c")
        s = lax.axis_index("s")

        pltpu.sync_copy(scales, sclv)

        def zf(r, carry):
            zv[r] = jnp.zeros((LANE,), jnp.float32)
            return carry
        lax.fori_loop(0, UB, zf, 0)

        rpt_all = stride // NS
        def zb(b, carry):
            pltpu.async_copy(zv, agg.at[pl.ds(s * rpt_all + b * UB, UB)], g4)
            return carry
        lax.fori_loop(0, rpt_all // UB, zb, 0)
        for _ in range(rpt_all // UB):
            pltpu.make_async_copy(zv, agg.at[pl.ds(0, UB)], g4).wait()
        plsc.subcore_barrier()

        def run_graph(gi, table, cidx, idsb, gath_o, n_rnd, e_pad):
            sc_vec = sclv[gi]
            nblk = e_pad // (NS * EB)
            rpt = n_rnd // NS
            ublk = rpt // UB

            for l in range(LAYERS):
                src = table if l == 0 else emb_o

                def sbody(i, carry):
                    for p, (cv, rb_, ssem, gs) in enumerate(banks):
                        b = 2 * i + p
                        offc = (s * nblk + b) * BLK

                        @pl.when(i >= 1)
                        def _drain():
                            for j in range(BLK):
                                pltpu.make_async_copy(
                                    rb_.at[j], agg.at[cv.at[j, 1]],
                                    ssem).wait()

                        pltpu.sync_copy(cidx.at[c].at[pl.ds(offc, BLK)], cv)
                        descs = [
                            pltpu.async_copy(src.at[cv.at[j, 0]], rb_.at[j],
                                             gs[j])
                            for j in range(BLK)]
                        for j in range(BLK):
                            descs[j].wait()
                            pltpu.async_copy(rb_.at[j], agg.at[cv.at[j, 1]],
                                             ssem, add=True)
                    return carry
                lax.fori_loop(0, nblk // 2, sbody, 0)
                for (cv, rb_, ssem, gs) in banks:
                    for j in range(BLK):
                        pltpu.make_async_copy(rb_.at[j], agg.at[cv.at[j, 1]],
                                              ssem).wait()
                plsc.subcore_barrier()

                def ubody(b, carry):
                    lo = s * rpt + b * UB
                    glo = c * stride + lo

                    @pl.when(b >= 1)
                    def _drain_stores():
                        pltpu.make_async_copy(
                            ev, emb_o.at[pl.ds(glo, UB)], g6).wait()
                        pltpu.make_async_copy(
                            sv, sum_o.at[pl.ds(glo, UB)], g7).wait()
                        pltpu.make_async_copy(
                            zv, agg.at[pl.ds(lo, UB)], g5).wait()

                    la = pltpu.async_copy(agg.at[pl.ds(lo, UB)], aggv, g0)
                    lb = pltpu.async_copy(src.at[pl.ds(glo, UB)], ev, g1)
                    if l > 0:
                        pltpu.async_copy(sum_o.at[pl.ds(glo, UB)], sv,
                                         g2).wait()
                    la.wait()
                    lb.wait()
                    pltpu.async_copy(zv, agg.at[pl.ds(lo, UB)], g5)

                    def rb(r, cc):
                        a = aggv[r]
                        e = ev[r]
                        nw = sc_vec * a + DECAY * e
                        ev[r] = nw
                        if l == 0:
                            sv[r] = e + nw
                        else:
                            sv[r] = sv[r] + nw
                        return cc
                    lax.fori_loop(0, UB, rb, 0)
                    pltpu.async_copy(ev, emb_o.at[pl.ds(glo, UB)], g6)
                    pltpu.async_copy(sv, sum_o.at[pl.ds(glo, UB)], g7)
                    return carry
                lax.fori_loop(0, ublk, ubody, 0)
                pltpu.make_async_copy(ev, emb_o.at[pl.ds(0, UB)], g6).wait()
                pltpu.make_async_copy(sv, sum_o.at[pl.ds(0, UB)], g7).wait()
                pltpu.make_async_copy(zv, agg.at[pl.ds(0, UB)], g5).wait()
                plsc.subcore_barrier()

            if idsb is not None:
                ipt = n_ids // NS // CHUNK
                for h in range(ipt // BLK):
                    cv, rb_, ssem, gs = banks[h % 2]
                    pltpu.sync_copy(idsb.at[c].at[s * (ipt // BLK) + h], cv)
                    descs = [
                        pltpu.async_copy(sum_o.at[cv.at[j, 0]], rb_.at[j],
                                         gs[j])
                        for j in range(BLK)]
                    for j in range(BLK):
                        descs[j].wait()
                        pltpu.sync_copy(
                            rb_.at[j],
                            gath_o.at[pl.ds(
                                c * n_ids + (s * ipt + h * BLK + j) * CHUNK,
                                CHUNK)])
                plsc.subcore_barrier()

        run_graph(0, t_s, ci_s, id_s, gath_s, nr_s, e_s)
        run_graph(1, t_e, ci_e, id_e, gath_e, nr_e, e_e)
        run_graph(2, t_k, ci_k, None, None, nr_k, e_k)

    return pl.kernel(
        body, out_type=out_type, scratch_types=scratch, mesh=mesh,
        compiler_params=pltpu.CompilerParams(use_tc_tiling_on_sc=False),
    ), stride, (nr_s, nr_e, nr_k)


def _prep_graph(table, row, col, n_nodes, n_rnd, stride):
    e = row.shape[0]
    e_pad = _ceil_to(e, 2 * NS * EB)
    tpad = jnp.pad(table, ((0, stride - n_nodes), (0, 0)))
    t2 = (tpad.reshape(stride, NC, HALF).transpose(1, 0, 2)
          .reshape(NC * stride, HALF))
    colp = jnp.pad(col, (0, e_pad - e))
    rowp = jnp.pad(row, (0, e_pad - e), constant_values=n_nodes)
    nch = e_pad // CHUNK
    cr = jnp.stack([jnp.stack([colp, colp + stride]),
                    jnp.stack([rowp, rowp])], axis=1)
    cidx = cr.reshape(NC, 2, nch, CHUNK).transpose(0, 2, 1, 3)
    return t2, cidx, e_pad


def _prep_ids(ids, stride):
    n_ids = ids.shape[0]
    nh = n_ids // (BLK * CHUNK)
    both = jnp.stack([ids, ids + stride])
    chunks = both.reshape(NC, nh, BLK, 1, CHUNK)
    return jnp.concatenate([chunks, jnp.zeros_like(chunks)], axis=3)


def _conv_all(s_tab, s_row, s_col, s_val, sid,
              e_tab, e_row, e_col, e_val, eid,
              k_tab, k_row, k_col, k_val):
    n_s, n_e, n_k = s_tab.shape[0], e_tab.shape[0], k_tab.shape[0]
    n_ids = sid.shape[0]
    e_s = _ceil_to(s_row.shape[0], 2 * NS * EB)
    e_e = _ceil_to(e_row.shape[0], 2 * NS * EB)
    e_k = _ceil_to(k_row.shape[0], 2 * NS * EB)
    fused, stride, (nr_s, nr_e, nr_k) = _make_fused(
        n_s, e_s, n_e, e_e, n_k, e_k, n_ids)

    t_s, ci_s, _ = _prep_graph(s_tab, s_row, s_col, n_s, nr_s, stride)
    t_e, ci_e, _ = _prep_graph(e_tab, e_row, e_col, n_e, nr_e, stride)
    t_k, ci_k, _ = _prep_graph(k_tab, k_row, k_col, n_k, nr_k, stride)
    id_s = _prep_ids(sid, stride)
    id_e = _prep_ids(eid, stride)
    scales = jnp.stack([
        jnp.broadcast_to(s_val[0], (LANE,)),
        jnp.broadcast_to(e_val[0], (LANE,)),
        jnp.broadcast_to(k_val[0], (LANE,)),
        jnp.zeros((LANE,), jnp.float32)])

    _, ssum, gs, ge = fused(t_s, ci_s, id_s, t_e, ci_e, id_e, t_k, ci_k,
                            scales)
    ems = gs.reshape(NC, n_ids, HALF).transpose(1, 0, 2).reshape(n_ids, 2 * HALF)
    eme = ge.reshape(NC, n_ids, HALF).transpose(1, 0, 2).reshape(n_ids, 2 * HALF)
    ck = (ssum.reshape(NC, stride, HALF)[:, :n_k]
          .transpose(1, 0, 2).reshape(n_k, 2 * HALF))
    return ems, eme, ck


RB = 2048


def _head(ems, eme, kn, ck, Ws, bS, We_, bE, Wk, bK, Wd, bD,
          W1, b1, W2, b2, W3, b3, W4, b4):
    b = ems.shape[0]
    k_num = ck.shape[0]

    def body(ems_r, eme_r, kn_r, ck_r, ws_r, bs_r, we_r, be_r, wk_r, bk_r,
             wd_r, bd_r, w1_r, b1_r, w2_r, b2_r, w3_r, b3_r, w4_r, b4_r,
             out_r):
        def dot_t(x, w):
            return lax.dot_general(x, w, (((1,), (1,)), ((), ())),
                                   preferred_element_type=jnp.float32)

        def lrelu(x):
            return jnp.where(x > 0, x, SLOPE * x)

        es = ems_r[...] * 0.25
        ee = eme_r[...] * 0.25
        ckv = ck_r[...] * 0.25
        sf = lrelu(dot_t(es, ws_r[...]) + bs_r[...])
        ef = lrelu(dot_t(ee, we_r[...]) + be_r[...])
        kf = lrelu(dot_t(ckv, wk_r[...]) + bk_r[...])
        disc = jax.nn.sigmoid(
            jnp.sum(ee * wd_r[...], axis=1, keepdims=True) + bd_r[...])
        st = disc * dot_t(sf - ef, kf) * kn_r[...]
        h = jnp.tanh(dot_t(st, w1_r[...]) + b1_r[...])
        h = jnp.tanh(dot_t(h, w2_r[...]) + b2_r[...])
        h = jnp.tanh(dot_t(h, w3_r[...]) + b3_r[...])
        o = jnp.sum(h * w4_r[...], axis=1, keepdims=True) + b4_r[...]
        out_r[...] = jax.nn.sigmoid(o)

    full = lambda shp: pl.BlockSpec(shp, lambda i: (0, 0))
    grid = b // RB
    return pl.pallas_call(
        body,
        grid=(grid,),
        in_specs=[
            pl.BlockSpec((RB, 2 * HALF), lambda i: (i, 0)),
            pl.BlockSpec((RB, 2 * HALF), lambda i: (i, 0)),
            pl.BlockSpec((RB, k_num), lambda i: (i, 0)),
            full(ck.shape), full(Ws.shape), full((1, Ws.shape[0])),
            full(We_.shape), full((1, We_.shape[0])),
            full(Wk.shape), full((1, Wk.shape[0])),
            full(Wd.shape), full((1, 1)),
            full(W1.shape), full((1, W1.shape[0])),
            full(W2.shape), full((1, W2.shape[0])),
            full(W3.shape), full((1, W3.shape[0])),
            full(W4.shape), full((1, 1)),
        ],
        out_specs=pl.BlockSpec((RB, 1), lambda i: (i, 0)),
        out_shape=jax.ShapeDtypeStruct((b, 1), jnp.float32),
    )(ems, eme, kn, ck, Ws, bS.reshape(1, -1), We_, bE.reshape(1, -1),
      Wk, bK.reshape(1, -1), Wd, bD.reshape(1, 1),
      W1, b1.reshape(1, -1), W2, b2.reshape(1, -1),
      W3, b3.reshape(1, -1), W4, b4.reshape(1, 1))


def kernel(student_id, exercise_id, knowledge,
           s_row, s_col, s_val, e_row, e_col, e_val, k_row, k_col, k_val,
           student_table, exercise_table, knowledge_table,
           Ws, bS, We_, bE, Wk, bK, Wd, bD,
           W1, b1, W2, b2, W3, b3, W4, b4):
    ems, eme, ck = _conv_all(student_table, s_row, s_col, s_val, student_id,
                             exercise_table, e_row, e_col, e_val, exercise_id,
                             knowledge_table, k_row, k_col, k_val)
    out = _head(ems, eme, knowledge, ck, Ws, bS, We_, bE, Wk, bK, Wd, bD,
                W1, b1, W2, b2, W3, b3, W4, b4)
    return out.reshape(-1)
